# baseline (speedup 1.0000x reference)
import functools

import jax
import jax.numpy as jnp
from jax import lax
from jax.experimental import pallas as pl
from jax.experimental.pallas import tpu as pltpu
from jax.experimental.pallas import tpu_sc as plsc

V = 1000
VP = 1024
B = 32
T = 1024
NTOK = B * T

_info = plsc.get_sparse_core_info()
NC = _info.num_cores
NS = _info.num_subcores
L = _info.num_lanes
NW = NC * NS
K = 4
TPC = NTOK // K
BPC = TPC // T
BPW = TPC // NW
CH = 32
NCH = BPW // CH
PK = 128


def _row_lse_body(tab_ref, out_ref):
    x = tab_ref[...]
    m = jnp.max(x, axis=1)
    s = jnp.sum(jnp.exp(x - m[:, None]), axis=1)
    out_ref[...] = jnp.log(s) + m


def _row_lse(table):
    return pl.pallas_call(
        _row_lse_body,
        out_shape=jax.ShapeDtypeStruct((V,), jnp.float32),
    )(table)


@functools.partial(
    pl.kernel,
    mesh=plsc.VectorSubcoreMesh(core_axis_name="c", subcore_axis_name="s"),
    compiler_params=pltpu.CompilerParams(
        use_tc_tiling_on_sc=True, needs_layout_passes=False),
    out_type=[
        jax.ShapeDtypeStruct((TPC, VP), jnp.float32),
        jax.ShapeDtypeStruct((NW, L), jnp.float32),
    ],
    scratch_types=[
        pltpu.VMEM((BPW,), jnp.int32),
        pltpu.VMEM((BPW,), jnp.int32),
        pltpu.VMEM((BPW,), jnp.int32),
        pltpu.VMEM((CH, VP), jnp.float32),
        pltpu.VMEM((CH, VP), jnp.float32),
        pltpu.VMEM((BPW,), jnp.float32),
        pltpu.VMEM((V,), jnp.float32),
        pltpu.VMEM((L,), jnp.float32),
        pltpu.SemaphoreType.DMA,
        pltpu.SemaphoreType.DMA,
        pltpu.SemaphoreType.DMA,
        pltpu.SemaphoreType.DMA,
        pltpu.SemaphoreType.DMA,
    ],
)
def _sc_gather(idx_hbm, tgt_hbm, lse_hbm, table_hbm, tflat_hbm,
               out_hbm, part_hbm,
               idx_v, tgt_v, fidx_v, rows_a, rows_b, pick_v, lse_v, acc_v,
               gsem_a, gsem_b, wsem_a, wsem_b, psem):
    wid = lax.axis_index("s") * NC + lax.axis_index("c")
    wbase = wid * BPW
    pltpu.sync_copy(idx_hbm.at[pl.ds(wbase, BPW)], idx_v)
    pltpu.sync_copy(tgt_hbm.at[pl.ds(wbase, BPW)], tgt_v)
    pltpu.sync_copy(lse_hbm, lse_v)

    def fidx_body(g, carry):
        sl = pl.ds(g * L, L)
        fidx_v[sl] = idx_v[sl] * VP + tgt_v[sl]
        return carry

    lax.fori_loop(0, BPW // L, fidx_body, 0)

    pick_cps = [
        pltpu.async_copy(tflat_hbm.at[fidx_v.at[pl.ds(j * PK, PK)]],
                         pick_v.at[pl.ds(j * PK, PK)], psem)
        for j in range(BPW // PK)
    ]

    def gather(c, buf, sem):
        pltpu.async_copy(table_hbm.at[idx_v.at[pl.ds(c * CH, CH)]], buf, sem)

    def writeout(c, buf, sem):
        pltpu.async_copy(buf, out_hbm.at[pl.ds(wbase + c * CH, CH)], sem)

    def gwait(buf, sem):
        pltpu.make_async_copy(table_hbm.at[pl.ds(0, CH)], buf, sem).wait()

    def wwait(buf, sem):
        pltpu.make_async_copy(buf, out_hbm.at[pl.ds(0, CH)], sem).wait()

    gather(0, rows_a, gsem_a)
    npairs = NCH // 2

    def pair_body(p, carry):
        c0 = p * 2
        gwait(rows_a, gsem_a)
        gather(c0 + 1, rows_b, gsem_b)
        writeout(c0, rows_a, wsem_a)
        gwait(rows_b, gsem_b)
        wwait(rows_a, wsem_a)

        @pl.when(p < npairs - 1)
        def _():
            gather(c0 + 2, rows_a, gsem_a)

        writeout(c0 + 1, rows_b, wsem_b)
        wwait(rows_b, wsem_b)
        return carry

    lax.fori_loop(0, npairs, pair_body, 0)

    for cp in pick_cps:
        cp.wait()
    acc_v[...] = jnp.full((L,), 0.0, jnp.float32)

    def loss_body(g, carry):
        sl = pl.ds(g * L, L)
        lse16 = plsc.load_gather(lse_v, [idx_v[sl]])
        acc_v[...] = acc_v[...] + (lse16 - pick_v[sl])
        return carry

    lax.fori_loop(0, BPW // L, loss_body, 0)
    pltpu.sync_copy(acc_v, part_hbm.at[wid])


def _copy_body(in_ref, out_ref):
    out_ref[...] = in_ref[...].T[None]


def _copy_alias_body(in_ref, prev_ref, out_ref):
    del prev_ref
    out_ref[...] = in_ref[...].T[None]


def _copy_chunk(k, chunk2d, prev):
    grid = (BPC, T // 128, VP // 128)
    in_spec = pl.BlockSpec((128, 128),
                           lambda bl, tt, vt: (bl * (T // 128) + tt, vt))
    out_spec = pl.BlockSpec((1, 128, 128),
                            lambda bl, tt, vt: (BPC * k + bl, vt, tt))
    out_shape = jax.ShapeDtypeStruct((B, V, T), jnp.float32)
    if prev is None:
        return pl.pallas_call(
            _copy_body, grid=grid, in_specs=[in_spec],
            out_specs=out_spec, out_shape=out_shape,
        )(chunk2d)
    return pl.pallas_call(
        _copy_alias_body, grid=grid,
        in_specs=[in_spec, pl.BlockSpec(memory_space=pl.ANY)],
        out_specs=out_spec, out_shape=out_shape,
        input_output_aliases={1: 0},
    )(chunk2d, prev)


def _finish_body(p_ref, out_ref):
    out_ref[...] = (jnp.sum(p_ref[...]) / jnp.float32(NTOK)).reshape(1, 1)


def _finish(partials):
    out = pl.pallas_call(
        _finish_body,
        out_shape=jax.ShapeDtypeStruct((1, 1), jnp.float32),
    )(partials)
    return out[0, 0]


def kernel(idx, targets, table):
    idx_f = idx.reshape(NTOK).astype(jnp.int32)
    tgt_f = targets.reshape(NTOK).astype(jnp.int32)
    table = table.astype(jnp.float32)
    table_p = jnp.pad(table, ((0, 0), (0, VP - V)))
    row_lse = _row_lse(table)
    tflat = jnp.concatenate(
        [table_p.reshape(V * VP), jnp.zeros(8, jnp.float32)])
    chunks = []
    parts = []
    for k in range(K):
        o, p = _sc_gather(idx_f[k * TPC:(k + 1) * TPC],
                          tgt_f[k * TPC:(k + 1) * TPC],
                          row_lse, table_p, tflat)
        chunks.append(o)
        parts.append(p)
    buf = None
    for k in range(K):
        buf = _copy_chunk(k, chunks[k], buf)
    loss = _finish(jnp.stack(parts).reshape(K * NW, L))
    logits = jnp.swapaxes(buf, 1, 2)
    return (logits, loss)


# speedup vs baseline: 5.4122x; 5.4122x over previous
import functools

import jax
import jax.numpy as jnp
from jax import lax
from jax.experimental import pallas as pl
from jax.experimental.pallas import tpu as pltpu
from jax.experimental.pallas import tpu_sc as plsc

V = 1000
VP = 1024
B = 32
T = 1024
NTOK = B * T

_info = plsc.get_sparse_core_info()
NC = _info.num_cores
NS = _info.num_subcores
L = _info.num_lanes
NW = NC * NS
BPW = NTOK // NW
CH = 32
NCH = BPW // CH
PK = 128


def _row_lse_body(tab_ref, out_ref):
    x = tab_ref[...]
    m = jnp.max(x, axis=1)
    s = jnp.sum(jnp.exp(x - m[:, None]), axis=1)
    out_ref[...] = jnp.log(s) + m


def _row_lse(table):
    return pl.pallas_call(
        _row_lse_body,
        out_shape=jax.ShapeDtypeStruct((V,), jnp.float32),
    )(table)


@functools.partial(
    pl.kernel,
    mesh=plsc.VectorSubcoreMesh(core_axis_name="c", subcore_axis_name="s"),
    compiler_params=pltpu.CompilerParams(
        use_tc_tiling_on_sc=True, needs_layout_passes=False),
    out_type=[
        jax.ShapeDtypeStruct((NTOK, VP), jnp.float32),
        jax.ShapeDtypeStruct((NW, L), jnp.float32),
    ],
    scratch_types=[
        pltpu.VMEM((BPW,), jnp.int32),
        pltpu.VMEM((BPW,), jnp.int32),
        pltpu.VMEM((BPW,), jnp.int32),
        pltpu.VMEM((CH, VP), jnp.float32),
        pltpu.VMEM((CH, VP), jnp.float32),
        pltpu.VMEM((BPW,), jnp.float32),
        pltpu.VMEM((V,), jnp.float32),
        pltpu.VMEM((L,), jnp.float32),
        pltpu.SemaphoreType.DMA,
        pltpu.SemaphoreType.DMA,
        pltpu.SemaphoreType.DMA,
        pltpu.SemaphoreType.DMA,
        pltpu.SemaphoreType.DMA,
    ],
)
def _sc_gather(idx_hbm, tgt_hbm, lse_hbm, table_hbm, tflat_hbm,
               out_hbm, part_hbm,
               idx_v, tgt_v, fidx_v, rows_a, rows_b, pick_v, lse_v, acc_v,
               gsem_a, gsem_b, wsem_a, wsem_b, psem):
    wid = lax.axis_index("s") * NC + lax.axis_index("c")
    wbase = wid * BPW
    pltpu.sync_copy(idx_hbm.at[pl.ds(wbase, BPW)], idx_v)
    pltpu.sync_copy(tgt_hbm.at[pl.ds(wbase, BPW)], tgt_v)
    pltpu.sync_copy(lse_hbm, lse_v)

    def fidx_body(g, carry):
        sl = pl.ds(g * L, L)
        fidx_v[sl] = idx_v[sl] * VP + tgt_v[sl]
        return carry

    lax.fori_loop(0, BPW // L, fidx_body, 0)

    pick_cps = [
        pltpu.async_copy(tflat_hbm.at[fidx_v.at[pl.ds(j * PK, PK)]],
                         pick_v.at[pl.ds(j * PK, PK)], psem)
        for j in range(BPW // PK)
    ]

    def gather(c, buf, sem):
        pltpu.async_copy(table_hbm.at[idx_v.at[pl.ds(c * CH, CH)]], buf, sem)

    def writeout(c, buf, sem):
        pltpu.async_copy(buf, out_hbm.at[pl.ds(wbase + c * CH, CH)], sem)

    def gwait(buf, sem):
        pltpu.make_async_copy(table_hbm.at[pl.ds(0, CH)], buf, sem).wait()

    def wwait(buf, sem):
        pltpu.make_async_copy(buf, out_hbm.at[pl.ds(0, CH)], sem).wait()

    gather(0, rows_a, gsem_a)
    npairs = NCH // 2

    def pair_body(p, carry):
        c0 = p * 2
        gwait(rows_a, gsem_a)
        gather(c0 + 1, rows_b, gsem_b)
        writeout(c0, rows_a, wsem_a)
        gwait(rows_b, gsem_b)
        wwait(rows_a, wsem_a)

        @pl.when(p < npairs - 1)
        def _():
            gather(c0 + 2, rows_a, gsem_a)

        writeout(c0 + 1, rows_b, wsem_b)
        wwait(rows_b, wsem_b)
        return carry

    lax.fori_loop(0, npairs, pair_body, 0)

    for cp in pick_cps:
        cp.wait()
    acc_v[...] = jnp.full((L,), 0.0, jnp.float32)

    def loss_body(g, carry):
        sl = pl.ds(g * L, L)
        lse16 = plsc.load_gather(lse_v, [idx_v[sl]])
        acc_v[...] = acc_v[...] + (lse16 - pick_v[sl])
        return carry

    lax.fori_loop(0, BPW // L, loss_body, 0)
    pltpu.sync_copy(acc_v, part_hbm.at[wid])


def _finish_body(p_ref, out_ref):
    out_ref[...] = (jnp.sum(p_ref[...]) / jnp.float32(NTOK)).reshape(1, 1)


def _finish(partials):
    out = pl.pallas_call(
        _finish_body,
        out_shape=jax.ShapeDtypeStruct((1, 1), jnp.float32),
    )(partials)
    return out[0, 0]


def kernel(idx, targets, table):
    idx_f = idx.reshape(NTOK).astype(jnp.int32)
    tgt_f = targets.reshape(NTOK).astype(jnp.int32)
    table = table.astype(jnp.float32)
    table_p = jnp.pad(table, ((0, 0), (0, VP - V)))
    row_lse = _row_lse(table)
    tflat = jnp.concatenate(
        [table_p.reshape(V * VP), jnp.zeros(8, jnp.float32)])
    out, parts = _sc_gather(idx_f, tgt_f, row_lse, table_p, tflat)
    loss = _finish(parts)
    logits = out.reshape(B, T, VP)[:, :, :V]
    return (logits, loss)


# speedup vs baseline: 5.4467x; 1.0064x over previous
import functools

import jax
import jax.numpy as jnp
from jax import lax
from jax.experimental import pallas as pl
from jax.experimental.pallas import tpu as pltpu
from jax.experimental.pallas import tpu_sc as plsc

V = 1000
VP = 1024
B = 32
T = 1024
NTOK = B * T

_info = plsc.get_sparse_core_info()
NC = _info.num_cores
NS = _info.num_subcores
L = _info.num_lanes
NW = NC * NS
BPW = NTOK // NW
CH = 32
NCH = BPW // CH
PK = 128


def _row_lse_body(tab_ref, out_ref):
    x = tab_ref[...]
    m = jnp.max(x, axis=1)
    s = jnp.sum(jnp.exp(x - m[:, None]), axis=1)
    out_ref[...] = jnp.log(s) + m


def _row_lse(table):
    return pl.pallas_call(
        _row_lse_body,
        out_shape=jax.ShapeDtypeStruct((V,), jnp.float32),
    )(table)


@functools.partial(
    pl.kernel,
    mesh=plsc.VectorSubcoreMesh(core_axis_name="c", subcore_axis_name="s"),
    compiler_params=pltpu.CompilerParams(
        use_tc_tiling_on_sc=True, needs_layout_passes=False),
    out_type=[
        jax.ShapeDtypeStruct((NTOK, VP), jnp.float32),
        jax.ShapeDtypeStruct((NW, L), jnp.float32),
    ],
    scratch_types=[
        pltpu.VMEM((BPW,), jnp.int32),
        pltpu.VMEM((BPW,), jnp.int32),
        pltpu.VMEM((BPW,), jnp.int32),
        pltpu.VMEM((CH, VP), jnp.float32),
        pltpu.VMEM((CH, VP), jnp.float32),
        pltpu.VMEM((BPW,), jnp.float32),
        pltpu.VMEM((V,), jnp.float32),
        pltpu.VMEM((L,), jnp.float32),
        pltpu.SemaphoreType.DMA,
        pltpu.SemaphoreType.DMA,
        pltpu.SemaphoreType.DMA,
        pltpu.SemaphoreType.DMA,
        pltpu.SemaphoreType.DMA,
    ],
)
def _sc_gather(idx_hbm, tgt_hbm, lse_hbm, table_hbm, tflat_hbm,
               out_hbm, part_hbm,
               idx_v, tgt_v, fidx_v, rows_a, rows_b, pick_v, lse_v, acc_v,
               gsem_a, gsem_b, wsem_a, wsem_b, psem):
    wid = lax.axis_index("s") * NC + lax.axis_index("c")
    wbase = wid * BPW
    pltpu.sync_copy(idx_hbm.at[wid], idx_v)
    pltpu.sync_copy(tgt_hbm.at[wid], tgt_v)
    pltpu.sync_copy(lse_hbm, lse_v)

    def fidx_body(g, carry):
        sl = pl.ds(g * L, L)
        fidx_v[sl] = idx_v[sl] * VP + tgt_v[sl]
        return carry

    lax.fori_loop(0, BPW // L, fidx_body, 0)

    pick_cps = [
        pltpu.async_copy(tflat_hbm.at[fidx_v.at[pl.ds(j * PK, PK)]],
                         pick_v.at[pl.ds(j * PK, PK)], psem)
        for j in range(BPW // PK)
    ]

    def gather(c, buf, sem):
        pltpu.async_copy(table_hbm.at[idx_v.at[pl.ds(c * CH, CH)]], buf, sem)

    def writeout(c, buf, sem):
        pltpu.async_copy(buf, out_hbm.at[pl.ds(wbase + c * CH, CH)], sem)

    def gwait(buf, sem):
        pltpu.make_async_copy(table_hbm.at[pl.ds(0, CH)], buf, sem).wait()

    def wwait(buf, sem):
        pltpu.make_async_copy(buf, out_hbm.at[pl.ds(0, CH)], sem).wait()

    gather(0, rows_a, gsem_a)
    npairs = NCH // 2

    def pair_body(p, carry):
        c0 = p * 2
        gwait(rows_a, gsem_a)
        gather(c0 + 1, rows_b, gsem_b)
        writeout(c0, rows_a, wsem_a)
        gwait(rows_b, gsem_b)
        wwait(rows_a, wsem_a)

        @pl.when(p < npairs - 1)
        def _():
            gather(c0 + 2, rows_a, gsem_a)

        writeout(c0 + 1, rows_b, wsem_b)
        wwait(rows_b, wsem_b)
        return carry

    lax.fori_loop(0, npairs, pair_body, 0)

    for cp in pick_cps:
        cp.wait()
    acc_v[...] = jnp.full((L,), 0.0, jnp.float32)

    def loss_body(g, carry):
        sl = pl.ds(g * L, L)
        lse16 = plsc.load_gather(lse_v, [idx_v[sl]])
        acc_v[...] = acc_v[...] + (lse16 - pick_v[sl])
        return carry

    lax.fori_loop(0, BPW // L, loss_body, 0)
    pltpu.sync_copy(acc_v, part_hbm.at[wid])


def _finish_body(p_ref, out_ref):
    out_ref[...] = (jnp.sum(p_ref[...]) / jnp.float32(NTOK)).reshape(1, 1)


def _finish(partials):
    out = pl.pallas_call(
        _finish_body,
        out_shape=jax.ShapeDtypeStruct((1, 1), jnp.float32),
    )(partials)
    return out[0, 0]


def kernel(idx, targets, table):
    idx_f = idx.astype(jnp.int32)
    tgt_f = targets.astype(jnp.int32)
    table = table.astype(jnp.float32)
    table_p = jnp.pad(table, ((0, 0), (0, VP - V)))
    row_lse = _row_lse(table)
    tflat = jnp.concatenate(
        [table_p.reshape(V * VP), jnp.zeros(8, jnp.float32)])
    out, parts = _sc_gather(idx_f, tgt_f, row_lse, table_p, tflat)
    loss = _finish(parts)
    logits = out.reshape(B, T, VP)[:, :, :V]
    return (logits, loss)


# speedup vs baseline: 5.5247x; 1.0143x over previous
import functools

import jax
import jax.numpy as jnp
from jax import lax
from jax.experimental import pallas as pl
from jax.experimental.pallas import tpu as pltpu
from jax.experimental.pallas import tpu_sc as plsc

V = 1000
VP = 1024
B = 32
T = 1024
NTOK = B * T

_info = plsc.get_sparse_core_info()
NC = _info.num_cores
NS = _info.num_subcores
L = _info.num_lanes
NW = NC * NS
BPW = NTOK // NW
CH = 16
NB = 4
NCH = BPW // CH
PK = 128


def _row_lse_body(tab_ref, out_ref):
    x = tab_ref[...]
    m = jnp.max(x, axis=1)
    s = jnp.sum(jnp.exp(x - m[:, None]), axis=1)
    out_ref[...] = jnp.log(s) + m


def _row_lse(table):
    return pl.pallas_call(
        _row_lse_body,
        out_shape=jax.ShapeDtypeStruct((V,), jnp.float32),
    )(table)


@functools.partial(
    pl.kernel,
    mesh=plsc.VectorSubcoreMesh(core_axis_name="c", subcore_axis_name="s"),
    compiler_params=pltpu.CompilerParams(
        use_tc_tiling_on_sc=True, needs_layout_passes=False),
    out_type=[
        jax.ShapeDtypeStruct((NTOK, VP), jnp.float32),
        jax.ShapeDtypeStruct((NW, L), jnp.float32),
    ],
    scratch_types=[
        pltpu.VMEM((BPW,), jnp.int32),
        pltpu.VMEM((BPW,), jnp.int32),
        pltpu.VMEM((BPW,), jnp.int32),
        pltpu.VMEM((CH, VP), jnp.float32),
        pltpu.VMEM((CH, VP), jnp.float32),
        pltpu.VMEM((CH, VP), jnp.float32),
        pltpu.VMEM((CH, VP), jnp.float32),
        pltpu.VMEM((BPW,), jnp.float32),
        pltpu.VMEM((V,), jnp.float32),
        pltpu.VMEM((L,), jnp.float32),
        pltpu.SemaphoreType.DMA,
        pltpu.SemaphoreType.DMA,
        pltpu.SemaphoreType.DMA,
        pltpu.SemaphoreType.DMA,
        pltpu.SemaphoreType.DMA,
        pltpu.SemaphoreType.DMA,
        pltpu.SemaphoreType.DMA,
        pltpu.SemaphoreType.DMA,
        pltpu.SemaphoreType.DMA,
    ],
)
def _sc_gather(idx_hbm, tgt_hbm, lse_hbm, table_hbm, tflat_hbm,
               out_hbm, part_hbm,
               idx_v, tgt_v, fidx_v, rows_0, rows_1, rows_2, rows_3,
               pick_v, lse_v, acc_v,
               gsem_0, gsem_1, gsem_2, gsem_3,
               wsem_0, wsem_1, wsem_2, wsem_3, psem):
    bufs = [rows_0, rows_1, rows_2, rows_3]
    gsems = [gsem_0, gsem_1, gsem_2, gsem_3]
    wsems = [wsem_0, wsem_1, wsem_2, wsem_3]
    wid = lax.axis_index("s") * NC + lax.axis_index("c")
    wbase = wid * BPW
    pltpu.sync_copy(idx_hbm.at[wid], idx_v)
    pltpu.sync_copy(tgt_hbm.at[wid], tgt_v)
    pltpu.sync_copy(lse_hbm, lse_v)

    def fidx_body(g, carry):
        sl = pl.ds(g * L, L)
        fidx_v[sl] = idx_v[sl] * VP + tgt_v[sl]
        return carry

    lax.fori_loop(0, BPW // L, fidx_body, 0)

    pick_cps = [
        pltpu.async_copy(tflat_hbm.at[fidx_v.at[pl.ds(j * PK, PK)]],
                         pick_v.at[pl.ds(j * PK, PK)], psem)
        for j in range(BPW // PK)
    ]

    def gather(c, s):
        pltpu.async_copy(table_hbm.at[idx_v.at[pl.ds(c * CH, CH)]],
                         bufs[s], gsems[s])

    def writeout(c, s):
        pltpu.async_copy(bufs[s], out_hbm.at[pl.ds(wbase + c * CH, CH)],
                         wsems[s])

    def gwait(s):
        pltpu.make_async_copy(table_hbm.at[pl.ds(0, CH)], bufs[s],
                              gsems[s]).wait()

    def wwait(s):
        pltpu.make_async_copy(bufs[s], out_hbm.at[pl.ds(0, CH)],
                              wsems[s]).wait()

    for s in range(NB):
        gather(s, s)
    nrounds = NCH // NB

    def round_body(r, carry):
        for s in range(NB):
            c = r * NB + s
            gwait(s)
            writeout(c, s)

            @pl.when(r < nrounds - 1)
            def _():
                wwait(s)
                gather(c + NB, s)

        return carry

    lax.fori_loop(0, nrounds, round_body, 0)
    for s in range(NB):
        wwait(s)

    for cp in pick_cps:
        cp.wait()
    acc_v[...] = jnp.full((L,), 0.0, jnp.float32)

    def loss_body(g, carry):
        sl = pl.ds(g * L, L)
        lse16 = plsc.load_gather(lse_v, [idx_v[sl]])
        acc_v[...] = acc_v[...] + (lse16 - pick_v[sl])
        return carry

    lax.fori_loop(0, BPW // L, loss_body, 0)
    pltpu.sync_copy(acc_v, part_hbm.at[wid])


def _finish_body(p_ref, out_ref):
    out_ref[...] = (jnp.sum(p_ref[...]) / jnp.float32(NTOK)).reshape(1, 1)


def _finish(partials):
    out = pl.pallas_call(
        _finish_body,
        out_shape=jax.ShapeDtypeStruct((1, 1), jnp.float32),
    )(partials)
    return out[0, 0]


def kernel(idx, targets, table):
    idx_f = idx.astype(jnp.int32)
    tgt_f = targets.astype(jnp.int32)
    table = table.astype(jnp.float32)
    table_p = jnp.pad(table, ((0, 0), (0, VP - V)))
    row_lse = _row_lse(table)
    tflat = jnp.concatenate(
        [table_p.reshape(V * VP), jnp.zeros(8, jnp.float32)])
    out, parts = _sc_gather(idx_f, tgt_f, row_lse, table_p, tflat)
    loss = _finish(parts)
    logits = out.reshape(B, T, VP)[:, :, :V]
    return (logits, loss)
